# Initial kernel scaffold; baseline (speedup 1.0000x reference)
import jax
import jax.numpy as jnp
from jax.experimental import pallas as pl


def kernel(X1, X2, X3, edge_index, W1, b1, W2, b2):
    raise NotImplementedError("write your pallas kernel here")


# speedup vs baseline: 17.0077x; 17.0077x over previous
import functools

import jax
import jax.numpy as jnp
from jax import lax
from jax.experimental import pallas as pl
from jax.experimental.pallas import tpu as pltpu
from jax.experimental.pallas import tpu_sc as plsc

N = 10000
E = 320000
D = 128

NC = 2
NS = 16
K = 80
EPT = E // (NC * NS)
NCHUNK = EPT // K
RPT = 640

_MESH = plsc.VectorSubcoreMesh(core_axis_name="c", subcore_axis_name="s")


def _over_rows(s, fn):
    @pl.when(s < NS - 1)
    def _():
        fn(pl.multiple_of(s * RPT, 8), RPT)

    @pl.when(s == NS - 1)
    def _():
        fn(N - 400, 400)


@functools.partial(
    pl.kernel,
    out_type=jax.ShapeDtypeStruct((NC, N, D), jnp.float32),
    mesh=_MESH,
    scratch_types=[
        pltpu.VMEM((NCHUNK, K), jnp.int32),
        pltpu.VMEM((K, D), jnp.float32),
        pltpu.VMEM_SHARED((N, D), jnp.float32),
    ],
)
def _deg_kernel(dst_hbm, ones_hbm, zeros_hbm, out_hbm, didx, ones_v, acc):
    c = lax.axis_index("c")
    s = lax.axis_index("s")

    pltpu.sync_copy(ones_hbm, ones_v)
    pltpu.sync_copy(dst_hbm.at[c, s], didx)
    _over_rows(s, lambda r0, nr: pltpu.sync_copy(
        zeros_hbm.at[pl.ds(r0, nr)], acc.at[pl.ds(r0, nr)]))
    plsc.subcore_barrier()

    def chunk(j, _):
        pltpu.sync_copy(ones_v, acc.at[didx.at[j]], add=True)
        return 0

    lax.fori_loop(0, NCHUNK, chunk, 0)
    plsc.subcore_barrier()
    _over_rows(s, lambda r0, nr: pltpu.sync_copy(
        acc.at[pl.ds(r0, nr)], out_hbm.at[c, pl.ds(r0, nr)]))


def _make_scatter(num_tables):
    def body(src_hbm, dst_hbm, *rest):
        tables = rest[:num_tables]
        zeros_hbm = rest[num_tables]
        out_hbm = rest[num_tables + 1]
        sidx, didx, rows, acc = rest[num_tables + 2:]
        c = lax.axis_index("c")
        s = lax.axis_index("s")

        pltpu.sync_copy(src_hbm.at[c, s], sidx)
        pltpu.sync_copy(dst_hbm.at[c, s], didx)

        for b in range(num_tables):
            g_hbm = tables[b]

            def init(r0, nr):
                @pl.when(c == 0)
                def _():
                    pltpu.sync_copy(g_hbm.at[pl.ds(r0, nr)],
                                    acc.at[pl.ds(r0, nr)])

                @pl.when(c != 0)
                def _():
                    pltpu.sync_copy(zeros_hbm.at[pl.ds(r0, nr)],
                                    acc.at[pl.ds(r0, nr)])

            _over_rows(s, init)
            plsc.subcore_barrier()

            def chunk(j, _):
                pltpu.sync_copy(g_hbm.at[sidx.at[j]], rows)
                pltpu.sync_copy(rows, acc.at[didx.at[j]], add=True)
                return 0

            lax.fori_loop(0, NCHUNK, chunk, 0)
            plsc.subcore_barrier()
            _over_rows(s, lambda r0, nr: pltpu.sync_copy(
                acc.at[pl.ds(r0, nr)], out_hbm.at[c, b, pl.ds(r0, nr)]))
            plsc.subcore_barrier()

    return pl.kernel(
        body,
        out_type=jax.ShapeDtypeStruct((NC, num_tables, N, D), jnp.float32),
        mesh=_MESH,
        scratch_types=[
            pltpu.VMEM((NCHUNK, K), jnp.int32),
            pltpu.VMEM((NCHUNK, K), jnp.int32),
            pltpu.VMEM((K, D), jnp.float32),
            pltpu.VMEM_SHARED((N, D), jnp.float32),
        ],
    )


_scatter3 = _make_scatter(3)
_scatter1 = _make_scatter(1)


_RB = 1000


def _dinv_block(degp_ref):
    deg = degp_ref[0, :, 0:1] + degp_ref[1, :, 0:1] + 1.0
    return lax.rsqrt(deg)


def _tc_pre_body(degp_ref, x1_ref, x2_ref, x3_ref, w1_ref,
                 g1_ref, g2_ref, g3_ref):
    dinv = _dinv_block(degp_ref)
    for x_ref, g_ref in ((x1_ref, g1_ref), (x2_ref, g2_ref), (x3_ref, g3_ref)):
        g_ref[...] = dinv * jnp.dot(x_ref[...], w1_ref[...],
                                    preferred_element_type=jnp.float32)


def _tc_mid_body(degp_ref, p_ref, b1_ref, w2_ref, zt_ref):
    dinv = _dinv_block(degp_ref)
    hsum = jnp.zeros((_RB, D), jnp.float32)
    for b in range(3):
        a = dinv * (p_ref[0, b] + p_ref[1, b]) + b1_ref[...]
        hsum = hsum + jnp.maximum(a, 0.0)
    zt_ref[...] = dinv * jnp.dot(hsum * (1.0 / 3.0), w2_ref[...],
                                 preferred_element_type=jnp.float32)


def _tc_post_body(degp_ref, q_ref, b2_ref, o_ref):
    dinv = _dinv_block(degp_ref)
    o_ref[...] = dinv * (q_ref[0, 0] + q_ref[1, 0]) + b2_ref[...]


def _row_spec(shape_prefix=()):
    nd = len(shape_prefix)
    return pl.BlockSpec(shape_prefix + (_RB, D),
                        lambda i: (0,) * nd + (i, 0))


_DEGP_SPEC = pl.BlockSpec((NC, _RB, D), lambda i: (0, i, 0))
_W_SPEC = pl.BlockSpec((D, D), lambda i: (0, 0))
_B_SPEC = pl.BlockSpec((1, D), lambda i: (0, 0))

_tc_pre = pl.pallas_call(
    _tc_pre_body,
    grid=(N // _RB,),
    in_specs=[_DEGP_SPEC, _row_spec(), _row_spec(), _row_spec(), _W_SPEC],
    out_specs=[_row_spec(), _row_spec(), _row_spec()],
    out_shape=[jax.ShapeDtypeStruct((N, D), jnp.float32)] * 3,
)

_tc_mid = pl.pallas_call(
    _tc_mid_body,
    grid=(N // _RB,),
    in_specs=[_DEGP_SPEC, _row_spec((NC, 3)), _B_SPEC, _W_SPEC],
    out_specs=_row_spec(),
    out_shape=jax.ShapeDtypeStruct((N, D), jnp.float32),
)

_tc_post = pl.pallas_call(
    _tc_post_body,
    grid=(N // _RB,),
    in_specs=[_DEGP_SPEC, _row_spec((NC, 1)), _B_SPEC],
    out_specs=_row_spec(),
    out_shape=jax.ShapeDtypeStruct((N, D), jnp.float32),
)


def kernel(X1, X2, X3, edge_index, W1, b1, W2, b2):
    src = edge_index[0].reshape(NC, NS, NCHUNK, K)
    dst = edge_index[1].reshape(NC, NS, NCHUNK, K)
    zeros_nd = jnp.zeros((N, D), jnp.float32)
    ones_kd = jnp.ones((K, D), jnp.float32)

    degp = _deg_kernel(dst, ones_kd, zeros_nd)
    g1, g2, g3 = _tc_pre(degp, X1, X2, X3, W1)
    p = _scatter3(src, dst, g1, g2, g3, zeros_nd)
    zt = _tc_mid(degp, p, b1.reshape(1, D), W2)
    q = _scatter1(src, dst, zt, zeros_nd)
    return _tc_post(degp, q, b2.reshape(1, D))


# speedup vs baseline: 24.7960x; 1.4579x over previous
import functools

import jax
import jax.numpy as jnp
from jax import lax
from jax.experimental import pallas as pl
from jax.experimental.pallas import tpu as pltpu
from jax.experimental.pallas import tpu_sc as plsc

N = 10000
E = 320000
D = 128

NC = 2
NS = 16
K = 80
EPT = E // (NC * NS)
NCHUNK = EPT // K
NB = 25
NBLK = NCHUNK // NB
RPT = 640

_MESH = plsc.VectorSubcoreMesh(core_axis_name="c", subcore_axis_name="s")


def _over_rows(s, fn):
    @pl.when(s < NS - 1)
    def _():
        fn(pl.multiple_of(s * RPT, 8), RPT)

    @pl.when(s == NS - 1)
    def _():
        fn(N - 400, 400)


@functools.partial(
    pl.kernel,
    out_type=jax.ShapeDtypeStruct((NC, N, D), jnp.float32),
    mesh=_MESH,
    scratch_types=[
        pltpu.VMEM((NB, K), jnp.int32),
        pltpu.VMEM((K, D), jnp.float32),
        pltpu.VMEM_SHARED((N, D), jnp.float32),
    ],
)
def _deg_kernel(dst_hbm, ones_hbm, zeros_hbm, out_hbm, didx, ones_v, acc):
    c = lax.axis_index("c")
    s = lax.axis_index("s")

    pltpu.sync_copy(ones_hbm, ones_v)
    _over_rows(s, lambda r0, nr: pltpu.sync_copy(
        zeros_hbm.at[pl.ds(r0, nr)], acc.at[pl.ds(r0, nr)]))
    plsc.subcore_barrier()

    def chunk(j, _):
        pltpu.sync_copy(ones_v, acc.at[didx.at[j]], add=True)
        return 0

    for blk in range(NBLK):
        pltpu.sync_copy(dst_hbm.at[c, s, blk], didx)
        lax.fori_loop(0, NB, chunk, 0)
    plsc.subcore_barrier()
    _over_rows(s, lambda r0, nr: pltpu.sync_copy(
        acc.at[pl.ds(r0, nr)], out_hbm.at[c, pl.ds(r0, nr)]))


def _make_scatter(num_tables):
    def body(src_hbm, dst_hbm, *rest):
        tables = rest[:num_tables]
        zeros_hbm = rest[num_tables]
        out_hbm = rest[num_tables + 1]
        sidx, didx, rows0, rows1, acc, sem0, sem1 = rest[num_tables + 2:]
        c = lax.axis_index("c")
        s = lax.axis_index("s")

        def wait_buf(buf, sem):
            pltpu.make_async_copy(zeros_hbm.at[pl.ds(0, K)], buf, sem).wait()

        for b in range(num_tables):
            g_hbm = tables[b]

            def init(r0, nr):
                @pl.when(c == 0)
                def _():
                    pltpu.sync_copy(g_hbm.at[pl.ds(r0, nr)],
                                    acc.at[pl.ds(r0, nr)])

                @pl.when(c != 0)
                def _():
                    pltpu.sync_copy(zeros_hbm.at[pl.ds(r0, nr)],
                                    acc.at[pl.ds(r0, nr)])

            _over_rows(s, init)
            plsc.subcore_barrier()

            for blk in range(NBLK):
                pltpu.sync_copy(src_hbm.at[c, s, blk], sidx)
                pltpu.sync_copy(dst_hbm.at[c, s, blk], didx)
                pltpu.async_copy(g_hbm.at[sidx.at[0]], rows0, sem0)

                def chunk2(j2, _):
                    j = j2 * 2
                    pltpu.async_copy(g_hbm.at[sidx.at[j + 1]], rows1, sem1)
                    wait_buf(rows0, sem0)
                    pltpu.sync_copy(rows0, acc.at[didx.at[j]], add=True)
                    pltpu.async_copy(g_hbm.at[sidx.at[j + 2]], rows0, sem0)
                    wait_buf(rows1, sem1)
                    pltpu.sync_copy(rows1, acc.at[didx.at[j + 1]], add=True)
                    return 0

                lax.fori_loop(0, (NB - 1) // 2, chunk2, 0)
                wait_buf(rows0, sem0)
                pltpu.sync_copy(rows0, acc.at[didx.at[NB - 1]], add=True)
            plsc.subcore_barrier()
            _over_rows(s, lambda r0, nr: pltpu.sync_copy(
                acc.at[pl.ds(r0, nr)], out_hbm.at[c, b, pl.ds(r0, nr)]))
            plsc.subcore_barrier()

    return pl.kernel(
        body,
        out_type=jax.ShapeDtypeStruct((NC, num_tables, N, D), jnp.float32),
        mesh=_MESH,
        scratch_types=[
            pltpu.VMEM((NB, K), jnp.int32),
            pltpu.VMEM((NB, K), jnp.int32),
            pltpu.VMEM((K, D), jnp.float32),
            pltpu.VMEM((K, D), jnp.float32),
            pltpu.VMEM_SHARED((N, D), jnp.float32),
            pltpu.SemaphoreType.DMA,
            pltpu.SemaphoreType.DMA,
        ],
    )


_scatter3 = _make_scatter(3)
_scatter1 = _make_scatter(1)


_RB = 1000


def _dinv_block(degp_ref):
    deg = degp_ref[0, :, 0:1] + degp_ref[1, :, 0:1] + 1.0
    return lax.rsqrt(deg)


def _tc_pre_body(degp_ref, x1_ref, x2_ref, x3_ref, w1_ref,
                 g1_ref, g2_ref, g3_ref):
    dinv = _dinv_block(degp_ref)
    for x_ref, g_ref in ((x1_ref, g1_ref), (x2_ref, g2_ref), (x3_ref, g3_ref)):
        g_ref[...] = dinv * jnp.dot(x_ref[...], w1_ref[...],
                                    preferred_element_type=jnp.float32)


def _tc_mid_body(degp_ref, p_ref, b1_ref, w2_ref, zt_ref):
    dinv = _dinv_block(degp_ref)
    hsum = jnp.zeros((_RB, D), jnp.float32)
    for b in range(3):
        a = dinv * (p_ref[0, b] + p_ref[1, b]) + b1_ref[...]
        hsum = hsum + jnp.maximum(a, 0.0)
    zt_ref[...] = dinv * jnp.dot(hsum * (1.0 / 3.0), w2_ref[...],
                                 preferred_element_type=jnp.float32)


def _tc_post_body(degp_ref, q_ref, b2_ref, o_ref):
    dinv = _dinv_block(degp_ref)
    o_ref[...] = dinv * (q_ref[0, 0] + q_ref[1, 0]) + b2_ref[...]


def _row_spec(shape_prefix=()):
    nd = len(shape_prefix)
    return pl.BlockSpec(shape_prefix + (_RB, D),
                        lambda i: (0,) * nd + (i, 0))


_DEGP_SPEC = pl.BlockSpec((NC, _RB, D), lambda i: (0, i, 0))
_W_SPEC = pl.BlockSpec((D, D), lambda i: (0, 0))
_B_SPEC = pl.BlockSpec((1, D), lambda i: (0, 0))

_tc_pre = pl.pallas_call(
    _tc_pre_body,
    grid=(N // _RB,),
    in_specs=[_DEGP_SPEC, _row_spec(), _row_spec(), _row_spec(), _W_SPEC],
    out_specs=[_row_spec(), _row_spec(), _row_spec()],
    out_shape=[jax.ShapeDtypeStruct((N, D), jnp.float32)] * 3,
)

_tc_mid = pl.pallas_call(
    _tc_mid_body,
    grid=(N // _RB,),
    in_specs=[_DEGP_SPEC, _row_spec((NC, 3)), _B_SPEC, _W_SPEC],
    out_specs=_row_spec(),
    out_shape=jax.ShapeDtypeStruct((N, D), jnp.float32),
)

_tc_post = pl.pallas_call(
    _tc_post_body,
    grid=(N // _RB,),
    in_specs=[_DEGP_SPEC, _row_spec((NC, 1)), _B_SPEC],
    out_specs=_row_spec(),
    out_shape=jax.ShapeDtypeStruct((N, D), jnp.float32),
)


def kernel(X1, X2, X3, edge_index, W1, b1, W2, b2):
    src = edge_index[0].reshape(NC, NS, NBLK, NB, K)
    dst = edge_index[1].reshape(NC, NS, NBLK, NB, K)
    zeros_nd = jnp.zeros((N, D), jnp.float32)
    ones_kd = jnp.ones((K, D), jnp.float32)

    degp = _deg_kernel(dst, ones_kd, zeros_nd)
    g1, g2, g3 = _tc_pre(degp, X1, X2, X3, W1)
    p = _scatter3(src, dst, g1, g2, g3, zeros_nd)
    zt = _tc_mid(degp, p, b1.reshape(1, D), W2)
    q = _scatter1(src, dst, zt, zeros_nd)
    return _tc_post(degp, q, b2.reshape(1, D))
